# baseline (speedup 1.0000x reference)
import functools
import jax
import jax.numpy as jnp
from jax import lax
from jax.experimental import pallas as pl
from jax.experimental.pallas import tpu as pltpu
from jax.experimental.pallas import tpu_sc as plsc

N = 10000
E = 320000
L = 4096
P = 10
EMB = 32
DUMP = 240
NPR = N + DUMP
ACC_ROWS = 3 * NPR
TILES = 32
E2 = 327680
EPT = E2 // TILES
C = 1024
NCH = EPT // C
ZR1 = NPR // 16

_mesh = plsc.VectorSubcoreMesh(core_axis_name="c", subcore_axis_name="s")


def _chunk_indices(j, dstv, wv, idxv, pkv, w1v, w2v, with_pk):
    iota = lax.iota(jnp.int32, 16)

    def body(k, _):
        off = k * 16
        d = dstv[pl.ds(off, 16)]
        if j > 0:
            m = wv[pl.ds(off, 16)] != 0.0
            dump = N + ((off + iota) & 127)
            idxv[pl.ds(off, 16)] = jnp.where(m, d, dump)
        if with_pk:
            m1 = w1v[pl.ds(off, 16)] != 0.0
            m2 = w2v[pl.ds(off, 16)] != 0.0
            pkv[pl.ds(off, 16)] = (d + jnp.where(m1, 65536, 0)
                                   + jnp.where(m2, 131072, 0))
        return _

    lax.fori_loop(0, C // 16, body, None)


def _seg_kernel(layer1, src_h, dst_h, w1_h, w2_h, t0_h, t1_h, t2_h, zblk_h,
                acc_out, pk_out, srcA, srcB, dstA, dstB, w1v, w2v,
                idxA, idxB, pkv, rowsA, rowsB, accs, semA, semB):
    c = lax.axis_index("c")
    s = lax.axis_index("s")
    wid = s * 2 + c
    base = wid * EPT
    tabs = [t0_h, t1_h, t2_h]
    srcb = [srcA, srcB]
    dstb = [dstA, dstB]
    idxb = [idxA, idxB]
    rowsb = [rowsA, rowsB]
    semb = [semA, semB]

    for j in range(3):
        with_pk = layer1 and j == 0
        wv = w1v if j == 1 else w2v

        def load_chunk(i, b):
            cb = base + i * C
            pltpu.sync_copy(src_h.at[pl.ds(cb, C)], srcb[b])
            pltpu.sync_copy(dst_h.at[pl.ds(cb, C)], dstb[b])
            if j == 1 or with_pk:
                pltpu.sync_copy(w1_h.at[pl.ds(cb, C)], w1v)
            if j == 2 or with_pk:
                pltpu.sync_copy(w2_h.at[pl.ds(cb, C)], w2v)
            _chunk_indices(j, dstb[b], wv, idxb[b], pkv, w1v, w2v, with_pk)
            if with_pk:
                pltpu.sync_copy(pkv, pk_out.at[pl.ds(cb, C)])
            return pltpu.async_copy(tabs[j].at[srcb[b]], rowsb[b], semb[b])

        pltpu.sync_copy(zblk_h, accs.at[pl.ds(s * ZR1, ZR1)])
        plsc.subcore_barrier()

        pending = {0: load_chunk(0, 0)}
        for i in range(NCH):
            b = i & 1
            if i + 1 < NCH:
                pending[(i + 1) & 1] = load_chunk(i + 1, b ^ 1)
            pending[b].wait()
            pltpu.sync_copy(rowsb[b], accs.at[dstb[b] if j == 0 else idxb[b]],
                            add=True)

        plsc.subcore_barrier()
        pltpu.sync_copy(accs.at[pl.ds(s * ZR1, ZR1)],
                        acc_out.at[c, pl.ds(j * NPR + s * ZR1, ZR1)])
        plsc.subcore_barrier()


def _make_seg(layer1):
    scratch = [
        pltpu.VMEM((C,), jnp.int32),
        pltpu.VMEM((C,), jnp.int32),
        pltpu.VMEM((C,), jnp.int32),
        pltpu.VMEM((C,), jnp.int32),
        pltpu.VMEM((C,), jnp.float32),
        pltpu.VMEM((C,), jnp.float32),
        pltpu.VMEM((C,), jnp.int32),
        pltpu.VMEM((C,), jnp.int32),
        pltpu.VMEM((C,), jnp.int32),
        pltpu.VMEM((C, EMB), jnp.float32),
        pltpu.VMEM((C, EMB), jnp.float32),
        pltpu.VMEM_SHARED((NPR, EMB), jnp.float32),
        pltpu.SemaphoreType.DMA,
        pltpu.SemaphoreType.DMA,
    ]
    acc_t = jax.ShapeDtypeStruct((2, ACC_ROWS, EMB), jnp.float32)
    if layer1:
        out_type = (acc_t, jax.ShapeDtypeStruct((E2,), jnp.int32))
        body = functools.partial(_seg_kernel, True)
    else:
        out_type = acc_t

        def body(src_h, dst_h, w1_h, w2_h, t0, t1, t2, zblk, acc_out, *scr):
            _seg_kernel(False, src_h, dst_h, w1_h, w2_h, t0, t1, t2, zblk,
                        acc_out, None, *scr)

    return pl.kernel(body, out_type=out_type, mesh=_mesh,
                     scratch_types=scratch,
                     compiler_params=pltpu.CompilerParams(
                         use_tc_tiling_on_sc=False),
                     name="seg1" if layer1 else "seg2")


_seg1 = _make_seg(True)
_seg2 = _make_seg(False)

LPW = L // TILES
LC = 32
LPC = LC * P
NCHL = LPW // LC


def _lnk_kernel(ic_h, ei_h, heads_h, rels_h, tails_h, dist_h, packed_h,
                nrep_h, labn_h, nfeat_h, rel_emb_h, dist_emb_h,
                nodesum_o, edgesum_o, hrep_o, trep_o, rrow_o, drow_o,
                hinit_o, tinit_o,
                icv, eiv, aiv, pkv, sgv, c1v, c2v, nrows, lrows,
                hv, tv, rv, dv, hr, tr, rr, dr, hi, ti, nsum, esum, sem):
    c = lax.axis_index("c")
    s = lax.axis_index("s")
    wid = s * 2 + c
    iota = lax.iota(jnp.int32, 16)
    perm = (iota + 8) & 15

    for k in range(NCHL):
        lbase = wid * LPW + k * LC
        fbase = lbase * P
        pltpu.sync_copy(ic_h.at[pl.ds(fbase, LPC)], icv)
        pltpu.sync_copy(ei_h.at[pl.ds(fbase, LPC)], eiv)
        pltpu.sync_copy(heads_h.at[pl.ds(lbase, LC)], hv)
        pltpu.sync_copy(tails_h.at[pl.ds(lbase, LC)], tv)
        pltpu.sync_copy(rels_h.at[pl.ds(lbase, LC)], rv)
        pltpu.sync_copy(dist_h.at[pl.ds(lbase, LC)], dv)

        def absn(q, _):
            off = q * 16
            aiv[pl.ds(off, 16)] = jnp.abs(icv[pl.ds(off, 16)])
            return _

        lax.fori_loop(0, LPC // 16, absn, None)
        pltpu.async_copy(nrep_h.at[aiv], nrows, sem).wait()

        def abse(q, _):
            off = q * 16
            aiv[pl.ds(off, 16)] = jnp.abs(eiv[pl.ds(off, 16)])
            return _

        lax.fori_loop(0, LPC // 16, abse, None)
        pltpu.async_copy(packed_h.at[aiv], pkv, sem).wait()

        def coef(q, _):
            off = q * 16
            pk = pkv[pl.ds(off, 16)]
            aiv[pl.ds(off, 16)] = pk & 0xFFFF
            n1 = 1.0 - ((pk >> 16) & 1).astype(jnp.float32)
            n2 = 1.0 - ((pk >> 17) & 1).astype(jnp.float32)
            es = jnp.where(eiv[pl.ds(off, 16)] != -1, 1.0, 0.0)
            den = jnp.maximum(n1 + n2, 1.0)
            c1v[pl.ds(off, 16)] = es * n1 / den
            c2v[pl.ds(off, 16)] = es * n2 / den
            sgv[pl.ds(off, 16)] = jnp.where(icv[pl.ds(off, 16)] != -1,
                                            1.0, 0.0)
            return _

        lax.fori_loop(0, LPC // 16, coef, None)
        pltpu.async_copy(labn_h.at[aiv], lrows, sem).wait()

        pltpu.async_copy(nrep_h.at[hv], hr, sem).wait()
        pltpu.async_copy(nrep_h.at[tv], tr, sem).wait()
        pltpu.async_copy(nfeat_h.at[hv], hi, sem).wait()
        pltpu.async_copy(nfeat_h.at[tv], ti, sem).wait()
        pltpu.async_copy(rel_emb_h.at[rv], rr, sem).wait()
        pltpu.async_copy(dist_emb_h.at[dv], dr, sem).wait()
        pltpu.sync_copy(hr, hrep_o.at[pl.ds(lbase, LC)])
        pltpu.sync_copy(tr, trep_o.at[pl.ds(lbase, LC)])
        pltpu.sync_copy(hi, hinit_o.at[pl.ds(lbase, LC)])
        pltpu.sync_copy(ti, tinit_o.at[pl.ds(lbase, LC)])
        pltpu.sync_copy(rr, rrow_o.at[pl.ds(lbase, LC)])
        pltpu.sync_copy(dr, drow_o.at[pl.ds(lbase, LC)])

        def link_body(i, _):
            f = i * P
            sgvec = sgv[pl.ds(f, 16)]
            c1vec = c1v[pl.ds(f, 16)]
            c2vec = c2v[pl.ds(f, 16)]
            accs = [jnp.zeros((16,), jnp.float32) for _ in range(4)]
            e8 = jnp.zeros((16,), jnp.float32)
            for p in range(P):
                sg = sgvec[p]
                for q in range(4):
                    accs[q] = accs[q] + nrows[f + p, pl.ds(q * 16, 16)] * sg
                v = lrows[f + p, pl.ds(0, 16)]
                vs = lax.gather(
                    v, perm[:, None],
                    lax.GatherDimensionNumbers(offset_dims=(),
                                               collapsed_slice_dims=(0,),
                                               start_index_map=(0,)),
                    slice_sizes=(1,),
                    mode=lax.GatherScatterMode.PROMISE_IN_BOUNDS)
                e8 = e8 + v * c1vec[p] + vs * c2vec[p]
            for q in range(4):
                nsum[i, pl.ds(q * 16, 16)] = accs[q]
            esum[i, pl.ds(0, 16)] = e8
            return _

        lax.fori_loop(0, LC, link_body, None)
        pltpu.sync_copy(nsum, nodesum_o.at[pl.ds(lbase, LC)])
        pltpu.sync_copy(esum, edgesum_o.at[pl.ds(lbase, LC)])


def _make_lnk():
    f32, i32 = jnp.float32, jnp.int32
    scratch = [
        pltpu.VMEM((LPC,), i32),
        pltpu.VMEM((LPC,), i32),
        pltpu.VMEM((LPC,), i32),
        pltpu.VMEM((LPC,), i32),
        pltpu.VMEM((LPC + 16,), f32),
        pltpu.VMEM((LPC + 16,), f32),
        pltpu.VMEM((LPC + 16,), f32),
        pltpu.VMEM((LPC, 64), f32),
        pltpu.VMEM((LPC, 16), f32),
        pltpu.VMEM((LC,), i32),
        pltpu.VMEM((LC,), i32),
        pltpu.VMEM((LC,), i32),
        pltpu.VMEM((LC,), i32),
        pltpu.VMEM((LC, 64), f32),
        pltpu.VMEM((LC, 64), f32),
        pltpu.VMEM((LC, 32), f32),
        pltpu.VMEM((LC, 32), f32),
        pltpu.VMEM((LC, 128), f32),
        pltpu.VMEM((LC, 128), f32),
        pltpu.VMEM((LC, 64), f32),
        pltpu.VMEM((LC, 16), f32),
        pltpu.SemaphoreType.DMA,
    ]
    out_type = (jax.ShapeDtypeStruct((L, 64), f32),
                jax.ShapeDtypeStruct((L, 16), f32),
                jax.ShapeDtypeStruct((L, 64), f32),
                jax.ShapeDtypeStruct((L, 64), f32),
                jax.ShapeDtypeStruct((L, 32), f32),
                jax.ShapeDtypeStruct((L, 32), f32),
                jax.ShapeDtypeStruct((L, 128), f32),
                jax.ShapeDtypeStruct((L, 128), f32))
    return pl.kernel(_lnk_kernel, out_type=out_type, mesh=_mesh,
                     scratch_types=scratch,
                     compiler_params=pltpu.CompilerParams(
                         use_tc_tiling_on_sc=False),
                     name="lnk")


_lnk = _make_lnk()


def _tca_body(x_ref, w_ref, o_ref):
    o_ref[...] = jnp.dot(x_ref[...], w_ref[...],
                         preferred_element_type=jnp.float32)


_tca = pl.pallas_call(
    _tca_body,
    out_shape=jax.ShapeDtypeStruct((N, EMB), jnp.float32))


def _tcb_body(acc_ref, b_ref, w_ref, p1_ref, h10_ref):
    for j in range(3):
        sl = pl.ds(j * NPR, N)
        x = acc_ref[0, sl, :] + acc_ref[1, sl, :]
        h = jnp.maximum(x + b_ref[...], 0.0)
        if j == 0:
            h10_ref[...] = h
        p1_ref[j] = jnp.dot(h, w_ref[...], preferred_element_type=jnp.float32)


_tcb = pl.pallas_call(
    _tcb_body,
    out_shape=(jax.ShapeDtypeStruct((3, N, EMB), jnp.float32),
               jax.ShapeDtypeStruct((N, EMB), jnp.float32)))


def _tcc_body(acc_ref, b_ref, h10_ref, wl_ref, bl_ref, nrep_ref, labn_ref):
    h2 = []
    for j in range(3):
        sl = pl.ds(j * NPR, N)
        x = acc_ref[0, sl, :] + acc_ref[1, sl, :]
        h2.append(jnp.maximum(x + b_ref[...], 0.0))
    nrep_ref[...] = jnp.concatenate([h10_ref[...], h2[0]], axis=1)
    la = [jnp.dot(h2[j], wl_ref[...], preferred_element_type=jnp.float32)
          + bl_ref[...] for j in (1, 2)]
    labn_ref[...] = jnp.concatenate(la, axis=1)


_tcc = pl.pallas_call(
    _tcc_body,
    out_shape=(jax.ShapeDtypeStruct((N, 64), jnp.float32),
               jax.ShapeDtypeStruct((N, 16), jnp.float32)))


def _tcd_body(nodesum_ref, edgesum_ref, ic_ref, drow_ref, hrep_ref, trep_ref,
              rrow_ref, hinit_ref, tinit_ref, wh_ref, bh_ref, wt_ref, bt_ref,
              wf1_ref, bf1_ref, wf2_ref, bf2_ref, wf3_ref, bf3_ref,
              x3_ref, hp_ref, tp_ref):
    def mm(a, b):
        return jnp.dot(a, b, preferred_element_type=jnp.float32)

    denom = jnp.clip(jnp.sum((ic_ref[...] != -1).astype(jnp.float32),
                             axis=1, keepdims=True), 1.0, None)
    node_mid = nodesum_ref[...] / denom
    edge_mid = edgesum_ref[:, :8] / denom
    drow = drow_ref[...]
    hp_ref[...] = (mm(node_mid, wh_ref[:64]) + mm(edge_mid, wh_ref[64:72])
                   + mm(drow, wh_ref[72:]) + bh_ref[...])
    tp_ref[...] = (mm(node_mid, wt_ref[:64]) + mm(edge_mid, wt_ref[64:72])
                   + mm(drow, wt_ref[72:]) + bt_ref[...])
    x = (mm(hrep_ref[...], wf1_ref[0:64]) + mm(trep_ref[...], wf1_ref[64:128])
         + mm(rrow_ref[...], wf1_ref[128:160]) + mm(node_mid, wf1_ref[160:224])
         + mm(edge_mid, wf1_ref[224:232]) + mm(hinit_ref[...], wf1_ref[232:360])
         + mm(tinit_ref[...], wf1_ref[360:488]) + bf1_ref[...])
    x = jnp.maximum(x, 0.0)
    x = jnp.maximum(mm(x, wf2_ref[...]) + bf2_ref[...], 0.0)
    x3_ref[...] = mm(x, wf3_ref[...]) + bf3_ref[...]


_tcd = pl.pallas_call(
    _tcd_body,
    out_shape=(jax.ShapeDtypeStruct((L, 128), jnp.float32),
               jax.ShapeDtypeStruct((L, 128), jnp.float32),
               jax.ShapeDtypeStruct((L, 128), jnp.float32)))


def kernel(node_feat, edge_index, links, dist, inter_count, edge_ids, rep_mask,
           W0, b0, W1, b1, rel_emb, dist_emb, Wl, bl, Wh, bh, Wt, bt,
           Wf1, bf1, Wf2, bf2, Wf3, bf3):
    pad = E2 - E
    src = jnp.concatenate([edge_index[0], jnp.zeros((pad,), jnp.int32)])
    dst = jnp.concatenate([edge_index[1],
                           N + (jnp.arange(pad, dtype=jnp.int32) & 127)])
    wz = jnp.zeros((pad,), jnp.float32)
    w1 = jnp.concatenate([rep_mask[:, 0], wz])
    w2 = jnp.concatenate([rep_mask[:, 1], wz])
    zblk = jnp.zeros((ZR1, EMB), jnp.float32)

    p0 = _tca(node_feat, W0)
    acc, packed = _seg1(src, dst, w1, w2, p0, p0, p0, zblk)
    p1, h10 = _tcb(acc, b0, W1)
    acc2 = _seg2(src, dst, w1, w2, p1[0], p1[1], p1[2], zblk)
    node_repr, labN = _tcc(acc2, b1, h10, Wl, bl)

    (nodesum, edgesum, head_repr, tail_repr, rel_rows, dist_rows,
     head_init, tail_init) = _lnk(
        inter_count.reshape(-1), edge_ids.reshape(-1), links[:, 0],
        links[:, 1], links[:, 2], dist, packed, node_repr, labN,
        node_feat, rel_emb, dist_emb)

    wf3p = jnp.pad(Wf3, ((0, 0), (0, 127)))
    x3, head_pred, tail_pred = _tcd(
        nodesum, edgesum, inter_count, dist_rows, head_repr, tail_repr,
        rel_rows, head_init, tail_init, Wh, bh, Wt, bt, Wf1, bf1,
        Wf2, bf2, wf3p, bf3)
    out = x3[:, :1]
    return (out, head_pred, tail_pred, head_init, tail_init)


# speedup vs baseline: 1.8140x; 1.8140x over previous
import functools
import jax
import jax.numpy as jnp
from jax import lax
from jax.experimental import pallas as pl
from jax.experimental.pallas import tpu as pltpu
from jax.experimental.pallas import tpu_sc as plsc

N = 10000
E = 320000
L = 4096
P = 10
EMB = 32
DUMP = 240
NPR = N + DUMP
ACC_ROWS = 3 * NPR
TILES = 32
E2 = E
EPT = E2 // TILES
C = 2000
NCH = EPT // C
ZR1 = NPR // 16

_mesh = plsc.VectorSubcoreMesh(core_axis_name="c", subcore_axis_name="s")


def _chunk_indices(j, dstv, wv, idxv, pkv, w1v, w2v, with_pk):
    iota = lax.iota(jnp.int32, 16)

    def body(k, _):
        off = k * 16
        d = dstv[pl.ds(off, 16)]
        if j > 0:
            m = wv[pl.ds(off, 16)] != 0.0
            dump = N + ((off + iota) & 127)
            idxv[pl.ds(off, 16)] = jnp.where(m, d, dump)
        if with_pk:
            m1 = w1v[pl.ds(off, 16)] != 0.0
            m2 = w2v[pl.ds(off, 16)] != 0.0
            pkv[pl.ds(off, 16)] = (d + jnp.where(m1, 65536, 0)
                                   + jnp.where(m2, 131072, 0))
        return _

    lax.fori_loop(0, C // 16, body, None)


def _seg_kernel(layer1, src_h, dst_h, w1_h, w2_h, t0_h, t1_h, t2_h, zblk_h,
                acc_out, pk_out, srcv, dstv, w1v, w2v, idxv, pkv, rows,
                accs, sem):
    c = lax.axis_index("c")
    s = lax.axis_index("s")
    wid = s * 2 + c
    base = wid * EPT
    tabs = [t0_h, t1_h, t2_h]

    for j in range(3):
        with_pk = layer1 and j == 0
        wv = w1v if j == 1 else w2v

        pltpu.sync_copy(zblk_h, accs.at[pl.ds(s * ZR1, ZR1)])
        plsc.subcore_barrier()

        for i in range(NCH):
            cb = base + i * C
            pltpu.sync_copy(src_h.at[pl.ds(cb, C)], srcv)
            pltpu.sync_copy(dst_h.at[pl.ds(cb, C)], dstv)
            if j == 1 or with_pk:
                pltpu.sync_copy(w1_h.at[pl.ds(cb, C)], w1v)
            if j == 2 or with_pk:
                pltpu.sync_copy(w2_h.at[pl.ds(cb, C)], w2v)
            _chunk_indices(j, dstv, wv, idxv, pkv, w1v, w2v, with_pk)
            if with_pk:
                pltpu.sync_copy(pkv, pk_out.at[pl.ds(cb, C)])
            pltpu.async_copy(tabs[j].at[srcv], rows, sem).wait()
            pltpu.sync_copy(rows, accs.at[dstv if j == 0 else idxv],
                            add=True)

        plsc.subcore_barrier()
        pltpu.sync_copy(accs.at[pl.ds(s * ZR1, ZR1)],
                        acc_out.at[c, pl.ds(j * NPR + s * ZR1, ZR1)])
        plsc.subcore_barrier()


def _make_seg(layer1):
    scratch = [
        pltpu.VMEM((C,), jnp.int32),
        pltpu.VMEM((C,), jnp.int32),
        pltpu.VMEM((C,), jnp.float32),
        pltpu.VMEM((C,), jnp.float32),
        pltpu.VMEM((C,), jnp.int32),
        pltpu.VMEM((C,), jnp.int32),
        pltpu.VMEM((C, EMB), jnp.float32),
        pltpu.VMEM_SHARED((NPR, EMB), jnp.float32),
        pltpu.SemaphoreType.DMA,
    ]
    acc_t = jax.ShapeDtypeStruct((2, ACC_ROWS, EMB), jnp.float32)
    if layer1:
        out_type = (acc_t, jax.ShapeDtypeStruct((E2,), jnp.int32))
        body = functools.partial(_seg_kernel, True)
    else:
        out_type = acc_t

        def body(src_h, dst_h, w1_h, w2_h, t0, t1, t2, zblk, acc_out, *scr):
            _seg_kernel(False, src_h, dst_h, w1_h, w2_h, t0, t1, t2, zblk,
                        acc_out, None, *scr)

    return pl.kernel(body, out_type=out_type, mesh=_mesh,
                     scratch_types=scratch,
                     compiler_params=pltpu.CompilerParams(
                         use_tc_tiling_on_sc=False),
                     name="seg1" if layer1 else "seg2")


_seg1 = _make_seg(True)
_seg2 = _make_seg(False)

LPW = L // TILES
LC = 32
LPC = LC * P
NCHL = LPW // LC


def _lnk_kernel(ic_h, ei_h, heads_h, rels_h, tails_h, dist_h, packed_h,
                nrep_h, labn_h, nfeat_h, rel_emb_h, dist_emb_h,
                nodesum_o, edgesum_o, hrep_o, trep_o, rrow_o, drow_o,
                hinit_o, tinit_o,
                icv, eiv, aiv, pkv, sgv, c1v, c2v, nrows, lrows,
                hv, tv, rv, dv, hr, tr, rr, dr, hi, ti, nsum, esum, sem):
    c = lax.axis_index("c")
    s = lax.axis_index("s")
    wid = s * 2 + c
    iota = lax.iota(jnp.int32, 16)
    perm = (iota + 8) & 15

    for k in range(NCHL):
        lbase = wid * LPW + k * LC
        fbase = lbase * P
        pltpu.sync_copy(ic_h.at[pl.ds(fbase, LPC)], icv)
        pltpu.sync_copy(ei_h.at[pl.ds(fbase, LPC)], eiv)
        pltpu.sync_copy(heads_h.at[pl.ds(lbase, LC)], hv)
        pltpu.sync_copy(tails_h.at[pl.ds(lbase, LC)], tv)
        pltpu.sync_copy(rels_h.at[pl.ds(lbase, LC)], rv)
        pltpu.sync_copy(dist_h.at[pl.ds(lbase, LC)], dv)

        def absn(q, _):
            off = q * 16
            aiv[pl.ds(off, 16)] = jnp.abs(icv[pl.ds(off, 16)])
            return _

        lax.fori_loop(0, LPC // 16, absn, None)
        pltpu.async_copy(nrep_h.at[aiv], nrows, sem).wait()

        def abse(q, _):
            off = q * 16
            aiv[pl.ds(off, 16)] = jnp.abs(eiv[pl.ds(off, 16)])
            return _

        lax.fori_loop(0, LPC // 16, abse, None)
        pltpu.async_copy(packed_h.at[aiv], pkv, sem).wait()

        def coef(q, _):
            off = q * 16
            pk = pkv[pl.ds(off, 16)]
            aiv[pl.ds(off, 16)] = pk & 0xFFFF
            n1 = 1.0 - ((pk >> 16) & 1).astype(jnp.float32)
            n2 = 1.0 - ((pk >> 17) & 1).astype(jnp.float32)
            es = jnp.where(eiv[pl.ds(off, 16)] != -1, 1.0, 0.0)
            den = jnp.maximum(n1 + n2, 1.0)
            c1v[pl.ds(off, 16)] = es * n1 / den
            c2v[pl.ds(off, 16)] = es * n2 / den
            sgv[pl.ds(off, 16)] = jnp.where(icv[pl.ds(off, 16)] != -1,
                                            1.0, 0.0)
            return _

        lax.fori_loop(0, LPC // 16, coef, None)
        pltpu.async_copy(labn_h.at[aiv], lrows, sem).wait()

        pltpu.async_copy(nrep_h.at[hv], hr, sem).wait()
        pltpu.async_copy(nrep_h.at[tv], tr, sem).wait()
        pltpu.async_copy(nfeat_h.at[hv], hi, sem).wait()
        pltpu.async_copy(nfeat_h.at[tv], ti, sem).wait()
        pltpu.async_copy(rel_emb_h.at[rv], rr, sem).wait()
        pltpu.async_copy(dist_emb_h.at[dv], dr, sem).wait()
        pltpu.sync_copy(hr, hrep_o.at[pl.ds(lbase, LC)])
        pltpu.sync_copy(tr, trep_o.at[pl.ds(lbase, LC)])
        pltpu.sync_copy(hi, hinit_o.at[pl.ds(lbase, LC)])
        pltpu.sync_copy(ti, tinit_o.at[pl.ds(lbase, LC)])
        pltpu.sync_copy(rr, rrow_o.at[pl.ds(lbase, LC)])
        pltpu.sync_copy(dr, drow_o.at[pl.ds(lbase, LC)])

        def link_body(i, _):
            f = i * P
            sgvec = sgv[pl.ds(f, 16)]
            c1vec = c1v[pl.ds(f, 16)]
            c2vec = c2v[pl.ds(f, 16)]
            accs = [jnp.zeros((16,), jnp.float32) for _ in range(4)]
            e8 = jnp.zeros((16,), jnp.float32)
            for p in range(P):
                sg = sgvec[p]
                for q in range(4):
                    accs[q] = accs[q] + nrows[f + p, pl.ds(q * 16, 16)] * sg
                v = lrows[f + p, pl.ds(0, 16)]
                vs = lax.gather(
                    v, perm[:, None],
                    lax.GatherDimensionNumbers(offset_dims=(),
                                               collapsed_slice_dims=(0,),
                                               start_index_map=(0,)),
                    slice_sizes=(1,),
                    mode=lax.GatherScatterMode.PROMISE_IN_BOUNDS)
                e8 = e8 + v * c1vec[p] + vs * c2vec[p]
            for q in range(4):
                nsum[i, pl.ds(q * 16, 16)] = accs[q]
            esum[i, pl.ds(0, 16)] = e8
            return _

        lax.fori_loop(0, LC, link_body, None)
        pltpu.sync_copy(nsum, nodesum_o.at[pl.ds(lbase, LC)])
        pltpu.sync_copy(esum, edgesum_o.at[pl.ds(lbase, LC)])


def _make_lnk():
    f32, i32 = jnp.float32, jnp.int32
    scratch = [
        pltpu.VMEM((LPC,), i32),
        pltpu.VMEM((LPC,), i32),
        pltpu.VMEM((LPC,), i32),
        pltpu.VMEM((LPC,), i32),
        pltpu.VMEM((LPC + 16,), f32),
        pltpu.VMEM((LPC + 16,), f32),
        pltpu.VMEM((LPC + 16,), f32),
        pltpu.VMEM((LPC, 64), f32),
        pltpu.VMEM((LPC, 16), f32),
        pltpu.VMEM((LC,), i32),
        pltpu.VMEM((LC,), i32),
        pltpu.VMEM((LC,), i32),
        pltpu.VMEM((LC,), i32),
        pltpu.VMEM((LC, 64), f32),
        pltpu.VMEM((LC, 64), f32),
        pltpu.VMEM((LC, 32), f32),
        pltpu.VMEM((LC, 32), f32),
        pltpu.VMEM((LC, 128), f32),
        pltpu.VMEM((LC, 128), f32),
        pltpu.VMEM((LC, 64), f32),
        pltpu.VMEM((LC, 16), f32),
        pltpu.SemaphoreType.DMA,
    ]
    out_type = (jax.ShapeDtypeStruct((L, 64), f32),
                jax.ShapeDtypeStruct((L, 16), f32),
                jax.ShapeDtypeStruct((L, 64), f32),
                jax.ShapeDtypeStruct((L, 64), f32),
                jax.ShapeDtypeStruct((L, 32), f32),
                jax.ShapeDtypeStruct((L, 32), f32),
                jax.ShapeDtypeStruct((L, 128), f32),
                jax.ShapeDtypeStruct((L, 128), f32))
    return pl.kernel(_lnk_kernel, out_type=out_type, mesh=_mesh,
                     scratch_types=scratch,
                     compiler_params=pltpu.CompilerParams(
                         use_tc_tiling_on_sc=False),
                     name="lnk")


_lnk = _make_lnk()


def _tca_body(x_ref, w_ref, o_ref):
    o_ref[...] = jnp.dot(x_ref[...], w_ref[...],
                         preferred_element_type=jnp.float32)


_tca = pl.pallas_call(
    _tca_body,
    out_shape=jax.ShapeDtypeStruct((N, EMB), jnp.float32))


def _tcb_body(acc_ref, b_ref, w_ref, p1_ref, h10_ref):
    for j in range(3):
        sl = pl.ds(j * NPR, N)
        x = acc_ref[0, sl, :] + acc_ref[1, sl, :]
        h = jnp.maximum(x + b_ref[...], 0.0)
        if j == 0:
            h10_ref[...] = h
        p1_ref[j] = jnp.dot(h, w_ref[...], preferred_element_type=jnp.float32)


_tcb = pl.pallas_call(
    _tcb_body,
    out_shape=(jax.ShapeDtypeStruct((3, N, EMB), jnp.float32),
               jax.ShapeDtypeStruct((N, EMB), jnp.float32)))


def _tcc_body(acc_ref, b_ref, h10_ref, wl_ref, bl_ref, nrep_ref, labn_ref):
    h2 = []
    for j in range(3):
        sl = pl.ds(j * NPR, N)
        x = acc_ref[0, sl, :] + acc_ref[1, sl, :]
        h2.append(jnp.maximum(x + b_ref[...], 0.0))
    nrep_ref[...] = jnp.concatenate([h10_ref[...], h2[0]], axis=1)
    la = [jnp.dot(h2[j], wl_ref[...], preferred_element_type=jnp.float32)
          + bl_ref[...] for j in (1, 2)]
    labn_ref[...] = jnp.concatenate(la, axis=1)


_tcc = pl.pallas_call(
    _tcc_body,
    out_shape=(jax.ShapeDtypeStruct((N, 64), jnp.float32),
               jax.ShapeDtypeStruct((N, 16), jnp.float32)))


def _tcd_body(nodesum_ref, edgesum_ref, ic_ref, drow_ref, hrep_ref, trep_ref,
              rrow_ref, hinit_ref, tinit_ref, wh_ref, bh_ref, wt_ref, bt_ref,
              wf1_ref, bf1_ref, wf2_ref, bf2_ref, wf3_ref, bf3_ref,
              x3_ref, hp_ref, tp_ref):
    def mm(a, b):
        return jnp.dot(a, b, preferred_element_type=jnp.float32)

    denom = jnp.clip(jnp.sum((ic_ref[...] != -1).astype(jnp.float32),
                             axis=1, keepdims=True), 1.0, None)
    node_mid = nodesum_ref[...] / denom
    edge_mid = edgesum_ref[:, :8] / denom
    drow = drow_ref[...]
    hp_ref[...] = (mm(node_mid, wh_ref[:64]) + mm(edge_mid, wh_ref[64:72])
                   + mm(drow, wh_ref[72:]) + bh_ref[...])
    tp_ref[...] = (mm(node_mid, wt_ref[:64]) + mm(edge_mid, wt_ref[64:72])
                   + mm(drow, wt_ref[72:]) + bt_ref[...])
    x = (mm(hrep_ref[...], wf1_ref[0:64]) + mm(trep_ref[...], wf1_ref[64:128])
         + mm(rrow_ref[...], wf1_ref[128:160]) + mm(node_mid, wf1_ref[160:224])
         + mm(edge_mid, wf1_ref[224:232]) + mm(hinit_ref[...], wf1_ref[232:360])
         + mm(tinit_ref[...], wf1_ref[360:488]) + bf1_ref[...])
    x = jnp.maximum(x, 0.0)
    x = jnp.maximum(mm(x, wf2_ref[...]) + bf2_ref[...], 0.0)
    x3_ref[...] = mm(x, wf3_ref[...]) + bf3_ref[...]


_tcd = pl.pallas_call(
    _tcd_body,
    out_shape=(jax.ShapeDtypeStruct((L, 128), jnp.float32),
               jax.ShapeDtypeStruct((L, 128), jnp.float32),
               jax.ShapeDtypeStruct((L, 128), jnp.float32)))


def kernel(node_feat, edge_index, links, dist, inter_count, edge_ids, rep_mask,
           W0, b0, W1, b1, rel_emb, dist_emb, Wl, bl, Wh, bh, Wt, bt,
           Wf1, bf1, Wf2, bf2, Wf3, bf3):
    src = edge_index[0]
    dst = edge_index[1]
    w1 = rep_mask[:, 0]
    w2 = rep_mask[:, 1]
    zblk = jnp.zeros((ZR1, EMB), jnp.float32)

    p0 = _tca(node_feat, W0)
    acc, packed = _seg1(src, dst, w1, w2, p0, p0, p0, zblk)
    p1, h10 = _tcb(acc, b0, W1)
    acc2 = _seg2(src, dst, w1, w2, p1[0], p1[1], p1[2], zblk)
    node_repr, labN = _tcc(acc2, b1, h10, Wl, bl)

    (nodesum, edgesum, head_repr, tail_repr, rel_rows, dist_rows,
     head_init, tail_init) = _lnk(
        inter_count.reshape(-1), edge_ids.reshape(-1), links[:, 0],
        links[:, 1], links[:, 2], dist, packed, node_repr, labN,
        node_feat, rel_emb, dist_emb)

    wf3p = jnp.pad(Wf3, ((0, 0), (0, 127)))
    x3, head_pred, tail_pred = _tcd(
        nodesum, edgesum, inter_count, dist_rows, head_repr, tail_repr,
        rel_rows, head_init, tail_init, Wh, bh, Wt, bt, Wf1, bf1,
        Wf2, bf2, wf3p, bf3)
    out = x3[:, :1]
    return (out, head_pred, tail_pred, head_init, tail_init)
